# Initial kernel scaffold; baseline (speedup 1.0000x reference)
import jax
import jax.numpy as jnp
from jax.experimental import pallas as pl


def kernel(x, edge_index, batch, W_in, b_in, Wg0, bg0, ln0_g, ln0_b, Wg1, bg1, ln1_g, ln1_b, W_out, b_out, lnout_g, lnout_b, W_graph, b_graph, lng_g, lng_b):
    raise NotImplementedError("write your pallas kernel here")


# speedup vs baseline: 11.0852x; 11.0852x over previous
import functools

import jax
import jax.numpy as jnp
from jax import lax
from jax.experimental import pallas as pl
from jax.experimental.pallas import tpu as pltpu
from jax.experimental.pallas import tpu_sc as plsc

NC = 2
NS = 16
NW = NC * NS
CHUNK = 128


def _elu(y):
    return jnp.where(y > 0, y, jnp.exp(y) - 1.0)


def _ln(t, g, b):
    mu = jnp.mean(t, axis=-1, keepdims=True)
    d = t - mu
    var = jnp.mean(d * d, axis=-1, keepdims=True)
    return d * lax.rsqrt(var + 1e-5) * g + b


def _make_deg_kernel(n_pad, cpt):
    rpt = n_pad // NS
    nz = rpt // CHUNK
    mesh = plsc.VectorSubcoreMesh(core_axis_name="c", subcore_axis_name="s")

    @functools.partial(
        pl.kernel,
        mesh=mesh,
        out_type=jax.ShapeDtypeStruct((NC, n_pad, 16), jnp.float32),
        scratch_types=[
            pltpu.VMEM((cpt, CHUNK), jnp.int32),
            pltpu.VMEM((CHUNK, 16), jnp.float32),
            pltpu.VMEM((CHUNK, 16), jnp.float32),
            pltpu.VMEM_SHARED((n_pad, 16), jnp.float32),
        ],
    )
    def deg_kernel(dst_hbm, out_hbm, idxd, zbuf, obuf, deg_sh):
        c = lax.axis_index("c")
        s = lax.axis_index("s")
        wid = c * NS + s
        pltpu.sync_copy(dst_hbm.at[pl.ds(wid * cpt, cpt)], idxd)

        def fill(i, _):
            zbuf[i, :] = jnp.zeros((16,), jnp.float32)
            obuf[i, :] = jnp.ones((16,), jnp.float32)
            return 0

        lax.fori_loop(0, CHUNK, fill, 0)
        for j in range(nz):
            pltpu.sync_copy(zbuf, deg_sh.at[pl.ds(s * rpt + j * CHUNK, CHUNK)])
        plsc.subcore_barrier()

        def body(k, _):
            pltpu.sync_copy(obuf, deg_sh.at[idxd.at[k]], add=True)
            return 0

        lax.fori_loop(0, cpt, body, 0)
        plsc.subcore_barrier()
        pltpu.sync_copy(deg_sh.at[pl.ds(s * rpt, rpt)],
                        out_hbm.at[c, pl.ds(s * rpt, rpt)])

    return deg_kernel


def _make_edge_kernel(n_pad, d, cpt):
    rpt = n_pad // NS
    nz = rpt // CHUNK
    half = cpt // 2
    mesh = plsc.VectorSubcoreMesh(core_axis_name="c", subcore_axis_name="s")

    @functools.partial(
        pl.kernel,
        mesh=mesh,
        out_type=jax.ShapeDtypeStruct((NC, n_pad, d), jnp.float32),
        scratch_types=[
            pltpu.VMEM((cpt + 8, CHUNK), jnp.int32),
            pltpu.VMEM((CHUNK,), jnp.int32),
            pltpu.VMEM((CHUNK,), jnp.int32),
            pltpu.VMEM((CHUNK, d), jnp.float32),
            pltpu.VMEM((CHUNK, d), jnp.float32),
            pltpu.VMEM_SHARED((n_pad, d), jnp.float32),
            pltpu.SemaphoreType.DMA,
            pltpu.SemaphoreType.DMA,
            pltpu.SemaphoreType.DMA,
            pltpu.SemaphoreType.DMA,
        ],
    )
    def edge_kernel(src_hbm, dst_hbm, p_hbm, out_hbm,
                    idxs, idxd0, idxd1, rows0, rows1, agg_sh,
                    gsem0, gsem1, dsem0, dsem1):
        c = lax.axis_index("c")
        s = lax.axis_index("s")
        wid = c * NS + s
        pltpu.sync_copy(src_hbm.at[pl.ds(wid * cpt, cpt + 8)], idxs)

        def zrow(i, _):
            def zcol(j, _):
                rows0[i, pl.ds(j * 16, 16)] = jnp.zeros((16,), jnp.float32)
                return 0
            lax.fori_loop(0, d // 16, zcol, 0)
            return 0

        lax.fori_loop(0, CHUNK, zrow, 0)
        for j in range(nz):
            pltpu.sync_copy(rows0, agg_sh.at[pl.ds(s * rpt + j * CHUNK, CHUNK)])
        plsc.subcore_barrier()

        ebase = wid * cpt * CHUNK

        pltpu.async_copy(p_hbm.at[idxs.at[0]], rows0, gsem0)
        pltpu.async_copy(dst_hbm.at[pl.ds(ebase, CHUNK)], idxd0, dsem0)

        def body(i, _):
            for b in (0, 1):
                k = 2 * i + b
                r_cur, gs_cur, i_cur, ds_cur = (
                    (rows0, gsem0, idxd0, dsem0) if b == 0
                    else (rows1, gsem1, idxd1, dsem1))
                r_nxt, gs_nxt, i_nxt, ds_nxt = (
                    (rows1, gsem1, idxd1, dsem1) if b == 0
                    else (rows0, gsem0, idxd0, dsem0))
                nxt = dst_hbm.at[pl.ds(ebase + (k + 1) * CHUNK, CHUNK)]
                pltpu.async_copy(p_hbm.at[idxs.at[k + 1]], r_nxt, gs_nxt)
                pltpu.async_copy(nxt, i_nxt, ds_nxt)
                pltpu.make_async_copy(p_hbm.at[idxs.at[k]], r_cur, gs_cur).wait()
                pltpu.make_async_copy(nxt, i_cur, ds_cur).wait()
                pltpu.sync_copy(r_cur, agg_sh.at[i_cur], add=True)
            return 0

        lax.fori_loop(0, half, body, 0)
        pltpu.make_async_copy(p_hbm.at[idxs.at[cpt]], rows0, gsem0).wait()
        pltpu.make_async_copy(dst_hbm.at[pl.ds(ebase, CHUNK)], idxd0,
                              dsem0).wait()
        plsc.subcore_barrier()
        pltpu.sync_copy(agg_sh.at[pl.ds(s * rpt, rpt)],
                        out_hbm.at[c, pl.ds(s * rpt, rpt)])

    return edge_kernel


def _tc1_body(x_ref, degp_ref, win_ref, bin_ref, wg0_ref, dis_ref, p0_ref):
    deg = degp_ref[0, :, 0:1] + degp_ref[1, :, 0:1] + 1.0
    dis = lax.rsqrt(deg)
    h = lax.dot_general(x_ref[...], win_ref[...], (((1,), (1,)), ((), ())),
                        preferred_element_type=jnp.float32)
    h = jnp.maximum(h + bin_ref[...], 0.0)
    p0 = lax.dot_general(h, wg0_ref[...], (((1,), (1,)), ((), ())),
                         preferred_element_type=jnp.float32)
    dis_ref[...] = dis
    p0_ref[...] = p0 * dis


def _tc2_body(aggp_ref, p_ref, dis_ref, bias_ref, g_ref, b_ref, w_ref, out_ref):
    dis = dis_ref[...]
    t = aggp_ref[0] + aggp_ref[1] + p_ref[...]
    t = t * dis + bias_ref[...]
    h = _elu(_ln(t, g_ref[...], b_ref[...]))
    p = lax.dot_general(h, w_ref[...], (((1,), (1,)), ((), ())),
                        preferred_element_type=jnp.float32)
    out_ref[...] = p * dis


def _tc3_body(aggp_ref, p_ref, dis_ref, batch_ref, bias_ref, g_ref, b_ref,
              wout_ref, bout_ref, log_ref, lob_ref,
              wg_ref, bg_ref, lgg_ref, lgb_ref,
              node_ref, graph_ref, acc_s, acc_c, *, nb, last_i):
    i = pl.program_id(0)
    dis = dis_ref[...]
    t = aggp_ref[0] + aggp_ref[1] + p_ref[...]
    t = t * dis + bias_ref[...]
    h = _elu(_ln(t, g_ref[...], b_ref[...]))

    ne = lax.dot_general(h, wout_ref[...], (((1,), (1,)), ((), ())),
                         preferred_element_type=jnp.float32)
    ne = jnp.maximum(ne + bout_ref[...], 0.0)
    node_ref[...] = _ln(ne, log_ref[...], lob_ref[...])

    ge = lax.dot_general(h, wg_ref[...], (((1,), (1,)), ((), ())),
                         preferred_element_type=jnp.float32)
    ge = jnp.maximum(ge + bg_ref[...], 0.0)
    ge = _ln(ge, lgg_ref[...], lgb_ref[...])

    r = ge.shape[0]
    oh_t = (batch_ref[...]
            == lax.broadcasted_iota(jnp.int32, (r, nb), 1)).astype(jnp.float32)

    @pl.when(i == 0)
    def _():
        acc_s[...] = jnp.zeros_like(acc_s)
        acc_c[...] = jnp.zeros_like(acc_c)

    acc_s[...] += lax.dot_general(oh_t, ge, (((0,), (0,)), ((), ())),
                                  preferred_element_type=jnp.float32)
    cnt = jnp.sum(oh_t, axis=0)[:, None]
    acc_c[...] += jnp.broadcast_to(cnt, acc_c.shape)

    @pl.when(i == last_i)
    def _():
        graph_ref[...] = acc_s[...] / jnp.maximum(acc_c[...], 1.0)


def kernel(x, edge_index, batch, W_in, b_in, Wg0, bg0, ln0_g, ln0_b,
           Wg1, bg1, ln1_g, ln1_b, W_out, b_out, lnout_g, lnout_b,
           W_graph, b_graph, lng_g, lng_b):
    n, d = x.shape
    e = edge_index.shape[1]
    nb = 64

    cpt = -(-e // (NW * CHUNK))
    cpt = -(-cpt // 8) * 8
    e_pad = NW * cpt * CHUNK
    nrows = e_pad // CHUNK + 8
    pad = nrows * CHUNK - e
    src2 = jnp.concatenate(
        [edge_index[0], jnp.zeros((pad,), jnp.int32)]).reshape(nrows, CHUNK)
    dst_flat = jnp.concatenate(
        [edge_index[1], jnp.full((pad,), n, jnp.int32)])
    dst2 = dst_flat.reshape(nrows, CHUNK)

    n_pad = NS * CHUNK * (-(-(n + 1) // (NS * CHUNK)))

    degp = _make_deg_kernel(n_pad, cpt)(dst2)
    edge_call = _make_edge_kernel(n_pad, d, cpt)

    r = 2000
    grid = (n // r,)
    last_i = grid[0] - 1
    row_spec = pl.BlockSpec((r, d), lambda i: (i, 0))
    vec_spec = pl.BlockSpec((r, 1), lambda i: (i, 0))
    par_spec = pl.BlockSpec((NC, r, d), lambda i: (0, i, 0))
    w_spec = pl.BlockSpec((d, d), lambda i: (0, 0))
    c_spec = pl.BlockSpec((d,), lambda i: (0,))

    dis, p0 = pl.pallas_call(
        _tc1_body,
        grid=grid,
        in_specs=[row_spec, pl.BlockSpec((NC, r, 16), lambda i: (0, i, 0)),
                  w_spec, c_spec, w_spec],
        out_specs=[vec_spec, row_spec],
        out_shape=[jax.ShapeDtypeStruct((n, 1), jnp.float32),
                   jax.ShapeDtypeStruct((n, d), jnp.float32)],
    )(x, degp, W_in, b_in, Wg0)

    aggp0 = edge_call(src2, dst_flat, p0)

    p1 = pl.pallas_call(
        _tc2_body,
        grid=grid,
        in_specs=[par_spec, row_spec, vec_spec, c_spec, c_spec, c_spec, w_spec],
        out_specs=row_spec,
        out_shape=jax.ShapeDtypeStruct((n, d), jnp.float32),
    )(aggp0, p0, dis, bg0, ln0_g, ln0_b, Wg1)

    aggp1 = edge_call(src2, dst_flat, p1)

    node_emb, graph_emb = pl.pallas_call(
        functools.partial(_tc3_body, nb=nb, last_i=last_i),
        grid=grid,
        in_specs=[par_spec, row_spec, vec_spec, vec_spec,
                  c_spec, c_spec, c_spec,
                  w_spec, c_spec, c_spec, c_spec,
                  w_spec, c_spec, c_spec, c_spec],
        out_specs=[row_spec, pl.BlockSpec((nb, d), lambda i: (0, 0))],
        out_shape=[jax.ShapeDtypeStruct((n, d), jnp.float32),
                   jax.ShapeDtypeStruct((nb, d), jnp.float32)],
        scratch_shapes=[pltpu.VMEM((nb, d), jnp.float32),
                        pltpu.VMEM((nb, d), jnp.float32)],
    )(aggp1, p1, dis, batch.reshape(n, 1), bg1, ln1_g, ln1_b,
      W_out, b_out, lnout_g, lnout_b, W_graph, b_graph, lng_g, lng_b)

    return (node_emb, graph_emb)
